# Initial kernel scaffold; baseline (speedup 1.0000x reference)
import functools

import jax
import jax.numpy as jnp
from jax import lax
from jax.experimental import pallas as pl
from jax.experimental.pallas import tpu as pltpu
from jax.experimental.pallas import tpu_sc as plsc

N = 10000
E = 160000
IN = 128
HID = 128
K = 10
SEQ = 4

NTILES = 32
ROWS = 320
NPAD = NTILES * ROWS
CH = 128
EPAD = ((E + CH - 1) // CH) * CH + CH
NOFF = 48


def _diffusion_body(feat_hbm, src_hbm, dstl_hbm, w_hbm, offs_hbm, out_hbm,
                    acc, rows, srcv, dlv, wv, wmv, offv, gsem, ssem):
    nc = 2
    wid = lax.axis_index("s") * nc + lax.axis_index("c")

    zero16 = jnp.zeros((16,), jnp.float32)

    def _zero_row(i, _):
        for j in range(HID // 16):
            acc[i, pl.ds(j * 16, 16)] = zero16
        return 0

    lax.fori_loop(0, ROWS, _zero_row, 0)

    pltpu.sync_copy(offs_hbm, offv)
    start = jnp.int32(0)
    end = jnp.int32(0)
    for g in range(NOFF // 16):
        lane = jax.lax.iota(jnp.int32, 16) + g * 16
        ov = offv[pl.ds(g * 16, 16)]
        start = jnp.maximum(start, jnp.max(jnp.where(lane == wid, ov, 0)))
        end = jnp.maximum(end, jnp.max(jnp.where(lane == wid + 1, ov, 0)))

    c0 = start // CH
    c1 = (end + CH - 1) // CH

    def _chunk(ci, _):
        base = ci * CH
        pltpu.sync_copy(src_hbm.at[pl.ds(base, CH)], srcv)
        pltpu.sync_copy(dstl_hbm.at[pl.ds(base, CH)], dlv)
        pltpu.sync_copy(w_hbm.at[pl.ds(base, CH)], wv)
        pltpu.async_copy(feat_hbm.at[srcv], rows, gsem).wait()
        for g in range(CH // 16):
            lane = base + g * 16 + jax.lax.iota(jnp.int32, 16)
            wvec = wv[pl.ds(g * 16, 16)]
            keep = (lane >= start) & (lane < end)
            wmv[pl.ds(g * 16, 16)] = jnp.where(keep, wvec, 0.0)

        def _scale(e, _):
            gbase = (e // 16) * 16
            wvec = wmv[pl.ds(gbase, 16)]
            wb = wvec[jnp.broadcast_to(e - gbase, (16,))]
            for j in range(HID // 16):
                rows[e, pl.ds(j * 16, 16)] = rows[e, pl.ds(j * 16, 16)] * wb
            return 0

        lax.fori_loop(0, CH, _scale, 0)
        pltpu.async_copy(rows, acc.at[dlv], ssem, add=True).wait()
        return 0

    lax.fori_loop(c0, c1, _chunk, 0)
    pltpu.sync_copy(acc, out_hbm.at[pl.ds(wid * ROWS, ROWS)])


@jax.jit
def _diffusion_step(feat, srcs, dstl, ws, offs):
    mesh = plsc.VectorSubcoreMesh(core_axis_name="c", subcore_axis_name="s")
    return pl.kernel(
        _diffusion_body,
        out_type=jax.ShapeDtypeStruct((NPAD, HID), jnp.float32),
        mesh=mesh,
        scratch_types=[
            pltpu.VMEM((ROWS, HID), jnp.float32),
            pltpu.VMEM((CH, HID), jnp.float32),
            pltpu.VMEM((CH,), jnp.int32),
            pltpu.VMEM((CH,), jnp.int32),
            pltpu.VMEM((CH,), jnp.float32),
            pltpu.VMEM((CH,), jnp.float32),
            pltpu.VMEM((NOFF,), jnp.int32),
            pltpu.SemaphoreType.DMA,
            pltpu.SemaphoreType.DMA,
        ],
    )(feat, srcs, dstl, ws, offs)


RBLK = 1280
GRID = NPAD // RBLK


def _px_body(wx_ref, b_ref, *refs):
    xs = refs[:K + 1]
    out = refs[K + 1]
    acc = jnp.broadcast_to(b_ref[0, :], (RBLK, HID))
    for k in range(K + 1):
        acc = acc + jnp.dot(xs[k][...], wx_ref[k],
                            preferred_element_type=jnp.float32)
    out[...] = acc


def _px_call(wx, b2, xs):
    blk = pl.BlockSpec((RBLK, HID), lambda i: (i, 0))
    return pl.pallas_call(
        _px_body,
        grid=(GRID,),
        in_specs=[pl.BlockSpec((K + 1, HID, HID), lambda i: (0, 0, 0)),
                  pl.BlockSpec((1, HID), lambda i: (0, 0))]
                 + [blk] * (K + 1),
        out_specs=blk,
        out_shape=jax.ShapeDtypeStruct((NPAD, HID), jnp.float32),
    )(wx, b2, *xs)


def _gate_body(wh_ref, px_ref, h_ref, *refs):
    hs = refs[:K + 1]
    ru_ref, rh_ref = refs[K + 1], refs[K + 2]
    acc = px_ref[...]
    for k in range(K + 1):
        acc = acc + jnp.dot(hs[k][...], wh_ref[k],
                            preferred_element_type=jnp.float32)
    ru = jax.nn.sigmoid(acc)
    ru_ref[...] = ru
    rh_ref[...] = ru * h_ref[...]


def _gate_call(wh, px, h, hs):
    blk = pl.BlockSpec((RBLK, HID), lambda i: (i, 0))
    return pl.pallas_call(
        _gate_body,
        grid=(GRID,),
        in_specs=[pl.BlockSpec((K + 1, HID, HID), lambda i: (0, 0, 0)),
                  blk, blk] + [blk] * (K + 1),
        out_specs=[blk, blk],
        out_shape=[jax.ShapeDtypeStruct((NPAD, HID), jnp.float32),
                   jax.ShapeDtypeStruct((NPAD, HID), jnp.float32)],
    )(wh, px, h, *hs)


def _cand_body(wh_ref, px_ref, h_ref, ru_ref, *refs):
    rhs = refs[:K + 1]
    out = refs[K + 1]
    acc = px_ref[...]
    for k in range(K + 1):
        acc = acc + jnp.dot(rhs[k][...], wh_ref[k],
                            preferred_element_type=jnp.float32)
    c = jax.nn.sigmoid(acc)
    ru = ru_ref[...]
    out[...] = ru * h_ref[...] + (1.0 - ru) * c


def _cand_call(wh, px, h, ru, rhs):
    blk = pl.BlockSpec((RBLK, HID), lambda i: (i, 0))
    return pl.pallas_call(
        _cand_body,
        grid=(GRID,),
        in_specs=[pl.BlockSpec((K + 1, HID, HID), lambda i: (0, 0, 0)),
                  blk, blk, blk] + [blk] * (K + 1),
        out_specs=blk,
        out_shape=jax.ShapeDtypeStruct((NPAD, HID), jnp.float32),
    )(wh, px, h, ru, *rhs)


def kernel(input, hidden, edge_index, edge_weight, W, b):
    src, dst = edge_index[0], edge_index[1]

    order = jnp.argsort(dst)
    dsts = dst[order]
    srcs = jnp.concatenate([src[order],
                            jnp.zeros((EPAD - E,), jnp.int32)])
    ws = jnp.concatenate([edge_weight[order],
                          jnp.zeros((EPAD - E,), jnp.float32)])
    dstl = jnp.concatenate([(dsts % ROWS).astype(jnp.int32),
                            jnp.zeros((EPAD - E,), jnp.int32)])
    bounds = (jnp.arange(NOFF, dtype=jnp.int32) * ROWS).clip(max=NPAD)
    offs = jnp.searchsorted(dsts, bounds).astype(jnp.int32)

    w3 = W.reshape(K + 1, IN + HID, HID)
    wx = w3[:, :IN, :]
    wh = w3[:, IN:, :]
    b2 = b.reshape(1, HID)

    pad_n = ((0, NPAD - N), (0, 0))
    xs_t = [jnp.pad(input[t], pad_n) for t in range(SEQ)]
    h = jnp.pad(hidden[0], pad_n)

    def chain(feat0):
        feats = [feat0]
        f = feat0
        for _ in range(K):
            f = _diffusion_step(f, srcs, dstl, ws, offs)
            feats.append(f)
        return feats

    pxs = [_px_call(wx, b2, chain(xs_t[t])) for t in range(SEQ)]

    outs = []
    for t in range(SEQ):
        hs = chain(h)
        ru, rh = _gate_call(wh, pxs[t], h, hs)
        rhs = chain(rh)
        h = _cand_call(wh, pxs[t], h, ru, rhs)
        outs.append(h[:N])

    output = jnp.stack(outs, axis=0)
    hidden_out = h[:N][None, :, :]
    return (output, hidden_out)


# speedup vs baseline: 3.2537x; 3.2537x over previous
import functools

import jax
import jax.numpy as jnp
from jax import lax
from jax.experimental import pallas as pl
from jax.experimental.pallas import tpu as pltpu
from jax.experimental.pallas import tpu_sc as plsc

N = 10000
E = 160000
IN = 128
HID = 128
K = 10
SEQ = 4

NTILES = 32
ROWS = 320
NPAD = NTILES * ROWS
CH = 128
EPAD = ((E + CH - 1) // CH) * CH + CH
NOFF = 48


def _diffusion_body(feat_hbm, src_hbm, dstl_hbm, w_hbm, offs_hbm, out_hbm,
                    acc, zbuf, rows, srcv, dlv, wv, wmv, offv, gsem, ssem):
    cid = lax.axis_index("c")
    sid = lax.axis_index("s")
    wid = cid * 16 + sid

    zero16 = jnp.zeros((16,), jnp.float32)

    def _zero_row(i, _):
        for j in range(HID // 16):
            zbuf[i, pl.ds(j * 16, 16)] = zero16
        return 0

    lax.fori_loop(0, ROWS, _zero_row, 0)
    pltpu.sync_copy(zbuf, acc.at[pl.ds(sid * ROWS, ROWS)])

    pltpu.sync_copy(offs_hbm, offv)
    ov = offv[pl.ds(wid, 16)]
    start = ov[0]
    end = ov[1]

    c0 = start // CH
    c1 = (end + CH - 1) // CH

    def _chunk(ci, _):
        base = ci * CH
        pltpu.sync_copy(src_hbm.at[pl.ds(base, CH)], srcv)
        pltpu.sync_copy(dstl_hbm.at[pl.ds(base, CH)], dlv)
        pltpu.sync_copy(w_hbm.at[pl.ds(base, CH)], wv)
        pltpu.async_copy(feat_hbm.at[srcv], rows, gsem).wait()
        for g in range(CH // 16):
            lane = base + g * 16 + jax.lax.iota(jnp.int32, 16)
            wvec = wv[pl.ds(g * 16, 16)]
            keep = (lane >= start) & (lane < end)
            wmv[pl.ds(g * 16, 16)] = jnp.where(keep, wvec, 0.0)

        def _scale(e, _):
            gbase = (e // 16) * 16
            wvec = wmv[pl.ds(gbase, 16)]
            wb = wvec[jnp.broadcast_to(e - gbase, (16,))]
            for j in range(HID // 16):
                rows[e, pl.ds(j * 16, 16)] = rows[e, pl.ds(j * 16, 16)] * wb
            return 0

        lax.fori_loop(0, CH, _scale, 0)
        pltpu.async_copy(rows, acc.at[dlv], ssem, add=True).wait()
        return 0

    lax.fori_loop(c0, c1, _chunk, 0)
    plsc.subcore_barrier()
    pltpu.sync_copy(acc.at[pl.ds(sid * ROWS, ROWS)],
                    out_hbm.at[pl.ds(wid * ROWS, ROWS)])


@jax.jit
def _diffusion_step(feat, srcs, dstl, ws, offs):
    mesh = plsc.VectorSubcoreMesh(core_axis_name="c", subcore_axis_name="s",
                                  num_cores=2, num_subcores=16)
    return pl.kernel(
        _diffusion_body,
        out_type=jax.ShapeDtypeStruct((NPAD, HID), jnp.float32),
        mesh=mesh,
        scratch_types=[
            pltpu.VMEM_SHARED((16 * ROWS, HID), jnp.float32),
            pltpu.VMEM((ROWS, HID), jnp.float32),
            pltpu.VMEM((CH, HID), jnp.float32),
            pltpu.VMEM((CH,), jnp.int32),
            pltpu.VMEM((CH,), jnp.int32),
            pltpu.VMEM((CH,), jnp.float32),
            pltpu.VMEM((CH,), jnp.float32),
            pltpu.VMEM((NOFF,), jnp.int32),
            pltpu.SemaphoreType.DMA,
            pltpu.SemaphoreType.DMA,
        ],
    )(feat, srcs, dstl, ws, offs)


RBLK = 1280
GRID = NPAD // RBLK


def _px_body(wx_ref, b_ref, x0_ref, xch_ref, out_ref):
    acc = jnp.broadcast_to(b_ref[0, :], (RBLK, HID))
    acc = acc + jnp.dot(x0_ref[...], wx_ref[0],
                        preferred_element_type=jnp.float32)
    for k in range(K):
        acc = acc + jnp.dot(xch_ref[k], wx_ref[k + 1],
                            preferred_element_type=jnp.float32)
    out_ref[...] = acc


def _px_call(wx, b2, x0, xch):
    blk = pl.BlockSpec((RBLK, HID), lambda i: (i, 0))
    chblk = pl.BlockSpec((K, RBLK, HID), lambda i: (0, i, 0))
    return pl.pallas_call(
        _px_body,
        grid=(GRID,),
        in_specs=[pl.BlockSpec((K + 1, HID, HID), lambda i: (0, 0, 0)),
                  pl.BlockSpec((1, HID), lambda i: (0, 0)), blk, chblk],
        out_specs=blk,
        out_shape=jax.ShapeDtypeStruct((NPAD, HID), jnp.float32),
    )(wx, b2, x0, xch)


def _gate_body(wh_ref, px_ref, h_ref, hch_ref, ru_ref, rh_ref):
    acc = px_ref[...]
    acc = acc + jnp.dot(h_ref[...], wh_ref[0],
                        preferred_element_type=jnp.float32)
    for k in range(K):
        acc = acc + jnp.dot(hch_ref[k], wh_ref[k + 1],
                            preferred_element_type=jnp.float32)
    ru = jax.nn.sigmoid(acc)
    ru_ref[...] = ru
    rh_ref[...] = ru * h_ref[...]


def _gate_call(wh, px, h, hch):
    blk = pl.BlockSpec((RBLK, HID), lambda i: (i, 0))
    chblk = pl.BlockSpec((K, RBLK, HID), lambda i: (0, i, 0))
    return pl.pallas_call(
        _gate_body,
        grid=(GRID,),
        in_specs=[pl.BlockSpec((K + 1, HID, HID), lambda i: (0, 0, 0)),
                  blk, blk, chblk],
        out_specs=[blk, blk],
        out_shape=[jax.ShapeDtypeStruct((NPAD, HID), jnp.float32),
                   jax.ShapeDtypeStruct((NPAD, HID), jnp.float32)],
    )(wh, px, h, hch)


def _cand_body(wh_ref, px_ref, h_ref, ru_ref, rh_ref, rhch_ref, out_ref):
    acc = px_ref[...]
    acc = acc + jnp.dot(rh_ref[...], wh_ref[0],
                        preferred_element_type=jnp.float32)
    for k in range(K):
        acc = acc + jnp.dot(rhch_ref[k], wh_ref[k + 1],
                            preferred_element_type=jnp.float32)
    c = jax.nn.sigmoid(acc)
    ru = ru_ref[...]
    out_ref[...] = ru * h_ref[...] + (1.0 - ru) * c


def _cand_call(wh, px, h, ru, rh, rhch):
    blk = pl.BlockSpec((RBLK, HID), lambda i: (i, 0))
    chblk = pl.BlockSpec((K, RBLK, HID), lambda i: (0, i, 0))
    return pl.pallas_call(
        _cand_body,
        grid=(GRID,),
        in_specs=[pl.BlockSpec((K + 1, HID, HID), lambda i: (0, 0, 0)),
                  blk, blk, blk, blk, chblk],
        out_specs=blk,
        out_shape=jax.ShapeDtypeStruct((NPAD, HID), jnp.float32),
    )(wh, px, h, ru, rh, rhch)


def kernel(input, hidden, edge_index, edge_weight, W, b):
    src, dst = edge_index[0], edge_index[1]

    order = jnp.argsort(dst)
    dsts = dst[order]
    srcs = jnp.concatenate([src[order],
                            jnp.zeros((EPAD - E,), jnp.int32)])
    ws = jnp.concatenate([edge_weight[order],
                          jnp.zeros((EPAD - E,), jnp.float32)])
    dstl = jnp.concatenate([(dsts % (16 * ROWS)).astype(jnp.int32),
                            jnp.zeros((EPAD - E,), jnp.int32)])
    bounds = (jnp.arange(NOFF, dtype=jnp.int32) * ROWS).clip(max=NPAD)
    offs = jnp.searchsorted(dsts, bounds).astype(jnp.int32)

    w3 = W.reshape(K + 1, IN + HID, HID)
    wx = w3[:, :IN, :]
    wh = w3[:, IN:, :]
    b2 = b.reshape(1, HID)

    pad_n = ((0, NPAD - N), (0, 0))
    xs4 = jnp.pad(input, ((0, 0),) + pad_n)
    h0 = jnp.pad(hidden[0], pad_n)

    def chain(feat0):
        def body(f, _):
            fn = _diffusion_step(f, srcs, dstl, ws, offs)
            return fn, fn
        _, ys = lax.scan(body, feat0, None, length=K)
        return ys

    def px_step(_, x0):
        xch = chain(x0)
        return 0, _px_call(wx, b2, x0, xch)

    _, pxs = lax.scan(px_step, 0, xs4)

    def tstep(h, px_t):
        hch = chain(h)
        ru, rh = _gate_call(wh, px_t, h, hch)
        rhch = chain(rh)
        hn = _cand_call(wh, px_t, h, ru, rh, rhch)
        return hn, hn

    h_fin, outs = lax.scan(tstep, h0, pxs)

    output = outs[:, :N, :]
    hidden_out = h_fin[:N][None, :, :]
    return (output, hidden_out)


# speedup vs baseline: 5.6691x; 1.7423x over previous
import functools

import jax
import jax.numpy as jnp
from jax import lax
from jax.experimental import pallas as pl
from jax.experimental.pallas import tpu as pltpu
from jax.experimental.pallas import tpu_sc as plsc

N = 10000
E = 160000
IN = 128
HID = 128
K = 10
SEQ = 4

NTILES = 32
ROWS = 320
NPAD = NTILES * ROWS
CH = 128
NCH = E // CH
NOFF = 48
NBUF = 3


def _diffusion_body(feat_hbm, edata_hbm, wdat_hbm, offs_hbm, out_hbm,
                    acc, rows0, rows1, rows2, ib0, ib1, ib2,
                    wb0, wb1, wb2, offv, g0, g1, g2, s0, s1, s2):
    cid = lax.axis_index("c")
    sid = lax.axis_index("s")
    wid = cid * 16 + sid
    rows = (rows0, rows1, rows2)
    ibs = (ib0, ib1, ib2)
    wbs = (wb0, wb1, wb2)
    gsem = (g0, g1, g2)
    ssem = (s0, s1, s2)

    zero16 = jnp.zeros((16,), jnp.float32)

    def _zero_row(i, _):
        for j in range(HID // 16):
            rows0[i, pl.ds(j * 16, 16)] = zero16
        return 0

    lax.fori_loop(0, CH, _zero_row, 0)
    abase = sid * ROWS
    pltpu.sync_copy(rows0, acc.at[pl.ds(abase, CH)])
    pltpu.sync_copy(rows0, acc.at[pl.ds(abase + CH, CH)])
    pltpu.sync_copy(rows0.at[pl.ds(0, ROWS - 2 * CH)],
                    acc.at[pl.ds(abase + 2 * CH, ROWS - 2 * CH)])

    pltpu.sync_copy(offs_hbm, offv)
    ov = offv[pl.ds(wid, 16)]
    start = ov[0]
    end = ov[1]

    c0 = start // CH
    c1 = (end + CH - 1) // CH
    n = c1 - c0

    def fetch(b, i):
        pltpu.sync_copy(edata_hbm.at[c0 + i], ibs[b])
        pltpu.sync_copy(wdat_hbm.at[c0 + i], wbs[b])
        pltpu.async_copy(feat_hbm.at[ibs[b].at[0]], rows[b], gsem[b])

    def consume(b, i):
        pltpu.make_async_copy(feat_hbm.at[ibs[b].at[0]], rows[b],
                              gsem[b]).wait()
        base = (c0 + i) * CH

        def _scale(e, _):
            gb = (e // 16) * 16
            wvec = wbs[b][pl.ds(gb, 16)]
            lane = base + gb + lax.iota(jnp.int32, 16)
            wvec = jnp.where((lane >= start) & (lane < end), wvec, 0.0)
            wb = wvec[jnp.broadcast_to(e - gb, (16,))]
            for j in range(HID // 16):
                rows[b][e, pl.ds(j * 16, 16)] = (
                    rows[b][e, pl.ds(j * 16, 16)] * wb)
            return 0

        lax.fori_loop(0, CH, _scale, 0)
        pltpu.async_copy(rows[b], acc.at[ibs[b].at[1]], ssem[b], add=True)

    def wait_scatter(b):
        pltpu.make_async_copy(rows[b], acc.at[ibs[b].at[1]], ssem[b]).wait()

    for b in range(2):
        @pl.when(b < n)
        def _(b=b):
            fetch(b, b)

    def body(jj, _):
        i0 = jj * NBUF
        for b in range(NBUF):
            i = i0 + b
            br = (b + 2) % NBUF

            @pl.when(i < n)
            def _(b=b, i=i, br=br):
                consume(b, i)
                k = i + 2

                @pl.when(k < n)
                def _():
                    @pl.when(k >= NBUF)
                    def _():
                        wait_scatter(br)
                    fetch(br, k)
        return 0

    lax.fori_loop(0, (n + NBUF - 1) // NBUF, body, 0)

    for b in range(NBUF):
        @pl.when(b < n)
        def _(b=b):
            wait_scatter(b)

    plsc.subcore_barrier()
    pltpu.sync_copy(acc.at[pl.ds(sid * ROWS, ROWS)],
                    out_hbm.at[pl.ds(wid * ROWS, ROWS)])


@jax.jit
def _diffusion_step(feat, edata, wdat, offs):
    mesh = plsc.VectorSubcoreMesh(core_axis_name="c", subcore_axis_name="s",
                                  num_cores=2, num_subcores=16)
    return pl.kernel(
        _diffusion_body,
        out_type=jax.ShapeDtypeStruct((NPAD, HID), jnp.float32),
        mesh=mesh,
        scratch_types=[
            pltpu.VMEM_SHARED((16 * ROWS, HID), jnp.float32),
            pltpu.VMEM((CH, HID), jnp.float32),
            pltpu.VMEM((CH, HID), jnp.float32),
            pltpu.VMEM((CH, HID), jnp.float32),
            pltpu.VMEM((2, CH), jnp.int32),
            pltpu.VMEM((2, CH), jnp.int32),
            pltpu.VMEM((2, CH), jnp.int32),
            pltpu.VMEM((CH,), jnp.float32),
            pltpu.VMEM((CH,), jnp.float32),
            pltpu.VMEM((CH,), jnp.float32),
            pltpu.VMEM((NOFF,), jnp.int32),
            pltpu.SemaphoreType.DMA,
            pltpu.SemaphoreType.DMA,
            pltpu.SemaphoreType.DMA,
            pltpu.SemaphoreType.DMA,
            pltpu.SemaphoreType.DMA,
            pltpu.SemaphoreType.DMA,
        ],
    )(feat, edata, wdat, offs)


RBLK = 1280
GRID = NPAD // RBLK


def _px_body(wx_ref, b_ref, x0_ref, xch_ref, out_ref):
    acc = jnp.broadcast_to(b_ref[0, :], (RBLK, HID))
    acc = acc + jnp.dot(x0_ref[...], wx_ref[0],
                        preferred_element_type=jnp.float32)
    for k in range(K):
        acc = acc + jnp.dot(xch_ref[k], wx_ref[k + 1],
                            preferred_element_type=jnp.float32)
    out_ref[...] = acc


def _px_call(wx, b2, x0, xch):
    blk = pl.BlockSpec((RBLK, HID), lambda i: (i, 0))
    chblk = pl.BlockSpec((K, RBLK, HID), lambda i: (0, i, 0))
    return pl.pallas_call(
        _px_body,
        grid=(GRID,),
        in_specs=[pl.BlockSpec((K + 1, HID, HID), lambda i: (0, 0, 0)),
                  pl.BlockSpec((1, HID), lambda i: (0, 0)), blk, chblk],
        out_specs=blk,
        out_shape=jax.ShapeDtypeStruct((NPAD, HID), jnp.float32),
    )(wx, b2, x0, xch)


def _gate_body(wh_ref, px_ref, h_ref, hch_ref, ru_ref, rh_ref):
    acc = px_ref[...]
    acc = acc + jnp.dot(h_ref[...], wh_ref[0],
                        preferred_element_type=jnp.float32)
    for k in range(K):
        acc = acc + jnp.dot(hch_ref[k], wh_ref[k + 1],
                            preferred_element_type=jnp.float32)
    ru = jax.nn.sigmoid(acc)
    ru_ref[...] = ru
    rh_ref[...] = ru * h_ref[...]


def _gate_call(wh, px, h, hch):
    blk = pl.BlockSpec((RBLK, HID), lambda i: (i, 0))
    chblk = pl.BlockSpec((K, RBLK, HID), lambda i: (0, i, 0))
    return pl.pallas_call(
        _gate_body,
        grid=(GRID,),
        in_specs=[pl.BlockSpec((K + 1, HID, HID), lambda i: (0, 0, 0)),
                  blk, blk, chblk],
        out_specs=[blk, blk],
        out_shape=[jax.ShapeDtypeStruct((NPAD, HID), jnp.float32),
                   jax.ShapeDtypeStruct((NPAD, HID), jnp.float32)],
    )(wh, px, h, hch)


def _cand_body(wh_ref, px_ref, h_ref, ru_ref, rh_ref, rhch_ref, out_ref):
    acc = px_ref[...]
    acc = acc + jnp.dot(rh_ref[...], wh_ref[0],
                        preferred_element_type=jnp.float32)
    for k in range(K):
        acc = acc + jnp.dot(rhch_ref[k], wh_ref[k + 1],
                            preferred_element_type=jnp.float32)
    c = jax.nn.sigmoid(acc)
    ru = ru_ref[...]
    out_ref[...] = ru * h_ref[...] + (1.0 - ru) * c


def _cand_call(wh, px, h, ru, rh, rhch):
    blk = pl.BlockSpec((RBLK, HID), lambda i: (i, 0))
    chblk = pl.BlockSpec((K, RBLK, HID), lambda i: (0, i, 0))
    return pl.pallas_call(
        _cand_body,
        grid=(GRID,),
        in_specs=[pl.BlockSpec((K + 1, HID, HID), lambda i: (0, 0, 0)),
                  blk, blk, blk, blk, chblk],
        out_specs=blk,
        out_shape=jax.ShapeDtypeStruct((NPAD, HID), jnp.float32),
    )(wh, px, h, ru, rh, rhch)


def kernel(input, hidden, edge_index, edge_weight, W, b):
    src, dst = edge_index[0], edge_index[1]

    order = jnp.argsort(dst)
    dsts = dst[order]
    srcs = src[order]
    wdat = edge_weight[order].reshape(NCH, CH)
    dstl = (dsts % (16 * ROWS)).astype(jnp.int32)
    edata = jnp.stack([srcs.reshape(NCH, CH), dstl.reshape(NCH, CH)], axis=1)
    bounds = (jnp.arange(NOFF, dtype=jnp.int32) * ROWS).clip(max=NPAD)
    offs = jnp.searchsorted(dsts, bounds).astype(jnp.int32)

    w3 = W.reshape(K + 1, IN + HID, HID)
    wx = w3[:, :IN, :]
    wh = w3[:, IN:, :]
    b2 = b.reshape(1, HID)

    pad_n = ((0, NPAD - N), (0, 0))
    xs4 = jnp.pad(input, ((0, 0),) + pad_n)
    h0 = jnp.pad(hidden[0], pad_n)

    def chain(feat0):
        def body(f, _):
            fn = _diffusion_step(f, edata, wdat, offs)
            return fn, fn
        _, ys = lax.scan(body, feat0, None, length=K)
        return ys

    def px_step(_, x0):
        xch = chain(x0)
        return 0, _px_call(wx, b2, x0, xch)

    _, pxs = lax.scan(px_step, 0, xs4)

    def tstep(h, px_t):
        hch = chain(h)
        ru, rh = _gate_call(wh, px_t, h, hch)
        rhch = chain(rh)
        hn = _cand_call(wh, px_t, h, ru, rh, rhch)
        return hn, hn

    h_fin, outs = lax.scan(tstep, h0, pxs)

    output = outs[:, :N, :]
    hidden_out = h_fin[:N][None, :, :]
    return (output, hidden_out)


# speedup vs baseline: 7.0378x; 1.2414x over previous
import functools

import jax
import jax.numpy as jnp
from jax import lax
from jax.experimental import pallas as pl
from jax.experimental.pallas import tpu as pltpu
from jax.experimental.pallas import tpu_sc as plsc

N = 10000
E = 160000
IN = 128
HID = 128
K = 10
SEQ = 4

NTILES = 32
ROWS = 320
NPAD = NTILES * ROWS
CH = 128
NCH = E // CH
NOFF = 48
NBUF = 3


def _diffusion_body(feat_hbm, edata_hbm, wdat_hbm, offs_hbm, out_hbm,
                    acc, rows0, rows1, rows2, ib0, ib1, ib2,
                    wb0, wb1, wb2, offv, g0, g1, g2, s0, s1, s2):
    cid = lax.axis_index("c")
    sid = lax.axis_index("s")
    wid = cid * 16 + sid
    rows = (rows0, rows1, rows2)
    ibs = (ib0, ib1, ib2)
    wbs = (wb0, wb1, wb2)
    gsem = (g0, g1, g2)
    ssem = (s0, s1, s2)

    zero16 = jnp.zeros((16,), jnp.float32)

    def _zero_row(i, _):
        for j in range(HID // 16):
            rows0[i, pl.ds(j * 16, 16)] = zero16
        return 0

    lax.fori_loop(0, CH, _zero_row, 0)
    abase = sid * ROWS
    pltpu.sync_copy(rows0, acc.at[pl.ds(abase, CH)])
    pltpu.sync_copy(rows0, acc.at[pl.ds(abase + CH, CH)])
    pltpu.sync_copy(rows0.at[pl.ds(0, ROWS - 2 * CH)],
                    acc.at[pl.ds(abase + 2 * CH, ROWS - 2 * CH)])

    pltpu.sync_copy(offs_hbm, offv)
    ov = offv[pl.ds(wid, 16)]
    start = ov[0]
    end = ov[1]

    c0 = start // CH
    c1 = (end + CH - 1) // CH
    n = c1 - c0

    def fetch(b, i):
        pltpu.sync_copy(edata_hbm.at[c0 + i], ibs[b])
        pltpu.sync_copy(wdat_hbm.at[c0 + i], wbs[b])
        pltpu.async_copy(feat_hbm.at[ibs[b].at[0]], rows[b], gsem[b])

    def consume(b, i):
        pltpu.make_async_copy(feat_hbm.at[ibs[b].at[0]], rows[b],
                              gsem[b]).wait()
        base = (c0 + i) * CH

        def _scale_group(g, _):
            gb = g * 16
            wvec = wbs[b][pl.ds(gb, 16)]
            lane = base + gb + lax.iota(jnp.int32, 16)
            wvec = jnp.where((lane >= start) & (lane < end), wvec, 0.0)
            for e in range(16):
                wb = wvec[jnp.broadcast_to(jnp.int32(e), (16,))]
                r = gb + e
                for j in range(HID // 16):
                    rows[b][r, pl.ds(j * 16, 16)] = (
                        rows[b][r, pl.ds(j * 16, 16)] * wb)
            return 0

        lax.fori_loop(0, CH // 16, _scale_group, 0)
        pltpu.async_copy(rows[b], acc.at[ibs[b].at[1]], ssem[b], add=True)

    def wait_scatter(b):
        pltpu.make_async_copy(rows[b], acc.at[ibs[b].at[1]], ssem[b]).wait()

    for b in range(2):
        @pl.when(b < n)
        def _(b=b):
            fetch(b, b)

    def body(jj, _):
        i0 = jj * NBUF
        for b in range(NBUF):
            i = i0 + b
            br = (b + 2) % NBUF

            @pl.when(i < n)
            def _(b=b, i=i, br=br):
                consume(b, i)
                k = i + 2

                @pl.when(k < n)
                def _():
                    @pl.when(k >= NBUF)
                    def _():
                        wait_scatter(br)
                    fetch(br, k)
        return 0

    lax.fori_loop(0, (n + NBUF - 1) // NBUF, body, 0)

    for b in range(NBUF):
        @pl.when(b < n)
        def _(b=b):
            wait_scatter(b)

    plsc.subcore_barrier()
    pltpu.sync_copy(acc.at[pl.ds(sid * ROWS, ROWS)],
                    out_hbm.at[pl.ds(wid * ROWS, ROWS)])


@jax.jit
def _diffusion_step(feat, edata, wdat, offs):
    mesh = plsc.VectorSubcoreMesh(core_axis_name="c", subcore_axis_name="s",
                                  num_cores=2, num_subcores=16)
    return pl.kernel(
        _diffusion_body,
        out_type=jax.ShapeDtypeStruct((NPAD, HID), jnp.float32),
        mesh=mesh,
        scratch_types=[
            pltpu.VMEM_SHARED((16 * ROWS, HID), jnp.float32),
            pltpu.VMEM((CH, HID), jnp.float32),
            pltpu.VMEM((CH, HID), jnp.float32),
            pltpu.VMEM((CH, HID), jnp.float32),
            pltpu.VMEM((2, CH), jnp.int32),
            pltpu.VMEM((2, CH), jnp.int32),
            pltpu.VMEM((2, CH), jnp.int32),
            pltpu.VMEM((CH,), jnp.float32),
            pltpu.VMEM((CH,), jnp.float32),
            pltpu.VMEM((CH,), jnp.float32),
            pltpu.VMEM((NOFF,), jnp.int32),
            pltpu.SemaphoreType.DMA,
            pltpu.SemaphoreType.DMA,
            pltpu.SemaphoreType.DMA,
            pltpu.SemaphoreType.DMA,
            pltpu.SemaphoreType.DMA,
            pltpu.SemaphoreType.DMA,
        ],
    )(feat, edata, wdat, offs)


RBLK = 1280
GRID = NPAD // RBLK


def _px_body(wx_ref, b_ref, x0_ref, xch_ref, out_ref):
    acc = jnp.broadcast_to(b_ref[0, :], (RBLK, HID))
    acc = acc + jnp.dot(x0_ref[...], wx_ref[0],
                        preferred_element_type=jnp.float32)
    for k in range(K):
        acc = acc + jnp.dot(xch_ref[k], wx_ref[k + 1],
                            preferred_element_type=jnp.float32)
    out_ref[...] = acc


def _px_call(wx, b2, x0, xch):
    blk = pl.BlockSpec((RBLK, HID), lambda i: (i, 0))
    chblk = pl.BlockSpec((K, RBLK, HID), lambda i: (0, i, 0))
    return pl.pallas_call(
        _px_body,
        grid=(GRID,),
        in_specs=[pl.BlockSpec((K + 1, HID, HID), lambda i: (0, 0, 0)),
                  pl.BlockSpec((1, HID), lambda i: (0, 0)), blk, chblk],
        out_specs=blk,
        out_shape=jax.ShapeDtypeStruct((NPAD, HID), jnp.float32),
    )(wx, b2, x0, xch)


def _gate_body(wh_ref, px_ref, h_ref, hch_ref, ru_ref, rh_ref):
    acc = px_ref[...]
    acc = acc + jnp.dot(h_ref[...], wh_ref[0],
                        preferred_element_type=jnp.float32)
    for k in range(K):
        acc = acc + jnp.dot(hch_ref[k], wh_ref[k + 1],
                            preferred_element_type=jnp.float32)
    ru = jax.nn.sigmoid(acc)
    ru_ref[...] = ru
    rh_ref[...] = ru * h_ref[...]


def _gate_call(wh, px, h, hch):
    blk = pl.BlockSpec((RBLK, HID), lambda i: (i, 0))
    chblk = pl.BlockSpec((K, RBLK, HID), lambda i: (0, i, 0))
    return pl.pallas_call(
        _gate_body,
        grid=(GRID,),
        in_specs=[pl.BlockSpec((K + 1, HID, HID), lambda i: (0, 0, 0)),
                  blk, blk, chblk],
        out_specs=[blk, blk],
        out_shape=[jax.ShapeDtypeStruct((NPAD, HID), jnp.float32),
                   jax.ShapeDtypeStruct((NPAD, HID), jnp.float32)],
    )(wh, px, h, hch)


def _cand_body(wh_ref, px_ref, h_ref, ru_ref, rh_ref, rhch_ref, out_ref):
    acc = px_ref[...]
    acc = acc + jnp.dot(rh_ref[...], wh_ref[0],
                        preferred_element_type=jnp.float32)
    for k in range(K):
        acc = acc + jnp.dot(rhch_ref[k], wh_ref[k + 1],
                            preferred_element_type=jnp.float32)
    c = jax.nn.sigmoid(acc)
    ru = ru_ref[...]
    out_ref[...] = ru * h_ref[...] + (1.0 - ru) * c


def _cand_call(wh, px, h, ru, rh, rhch):
    blk = pl.BlockSpec((RBLK, HID), lambda i: (i, 0))
    chblk = pl.BlockSpec((K, RBLK, HID), lambda i: (0, i, 0))
    return pl.pallas_call(
        _cand_body,
        grid=(GRID,),
        in_specs=[pl.BlockSpec((K + 1, HID, HID), lambda i: (0, 0, 0)),
                  blk, blk, blk, blk, chblk],
        out_specs=blk,
        out_shape=jax.ShapeDtypeStruct((NPAD, HID), jnp.float32),
    )(wh, px, h, ru, rh, rhch)


def kernel(input, hidden, edge_index, edge_weight, W, b):
    src, dst = edge_index[0], edge_index[1]

    order = jnp.argsort(dst)
    dsts = dst[order]
    srcs = src[order]
    wdat = edge_weight[order].reshape(NCH, CH)
    dstl = (dsts % (16 * ROWS)).astype(jnp.int32)
    edata = jnp.stack([srcs.reshape(NCH, CH), dstl.reshape(NCH, CH)], axis=1)
    bounds = (jnp.arange(NOFF, dtype=jnp.int32) * ROWS).clip(max=NPAD)
    offs = jnp.searchsorted(dsts, bounds).astype(jnp.int32)

    w3 = W.reshape(K + 1, IN + HID, HID)
    wx = w3[:, :IN, :]
    wh = w3[:, IN:, :]
    b2 = b.reshape(1, HID)

    pad_n = ((0, NPAD - N), (0, 0))
    xs4 = jnp.pad(input, ((0, 0),) + pad_n)
    h0 = jnp.pad(hidden[0], pad_n)

    def chain(feat0):
        def body(f, _):
            fn = _diffusion_step(f, edata, wdat, offs)
            return fn, fn
        _, ys = lax.scan(body, feat0, None, length=K)
        return ys

    def px_step(_, x0):
        xch = chain(x0)
        return 0, _px_call(wx, b2, x0, xch)

    _, pxs = lax.scan(px_step, 0, xs4)

    def tstep(h, px_t):
        hch = chain(h)
        ru, rh = _gate_call(wh, px_t, h, hch)
        rhch = chain(rh)
        hn = _cand_call(wh, px_t, h, ru, rh, rhch)
        return hn, hn

    h_fin, outs = lax.scan(tstep, h0, pxs)

    output = outs[:, :N, :]
    hidden_out = h_fin[:N][None, :, :]
    return (output, hidden_out)


# speedup vs baseline: 7.7726x; 1.1044x over previous
import functools

import jax
import jax.numpy as jnp
from jax import lax
from jax.experimental import pallas as pl
from jax.experimental.pallas import tpu as pltpu
from jax.experimental.pallas import tpu_sc as plsc

N = 10000
E = 160000
IN = 128
HID = 128
K = 10
SEQ = 4

NTILES = 32
ROWS = 320
NPAD = NTILES * ROWS
CH = 128
NCH = E // CH
NOFF = 48
NBUF = 3


def _diffusion_body(feat_hbm, edata_hbm, wdat_hbm, offs_hbm, out_hbm,
                    acc, rows0, rows1, rows2, ib0, ib1, ib2,
                    wb0, wb1, wb2, offv, g0, g1, g2, s0, s1, s2,
                    i0s, i1s, i2s):
    cid = lax.axis_index("c")
    sid = lax.axis_index("s")
    wid = cid * 16 + sid
    rows = (rows0, rows1, rows2)
    ibs = (ib0, ib1, ib2)
    wbs = (wb0, wb1, wb2)
    gsem = (g0, g1, g2)
    ssem = (s0, s1, s2)
    isem = (i0s, i1s, i2s)

    zero16 = jnp.zeros((16,), jnp.float32)

    def _zero_row(i, _):
        for j in range(HID // 16):
            rows0[i, pl.ds(j * 16, 16)] = zero16
        return 0

    lax.fori_loop(0, CH, _zero_row, 0)
    abase = sid * ROWS
    pltpu.sync_copy(rows0, acc.at[pl.ds(abase, CH)])
    pltpu.sync_copy(rows0, acc.at[pl.ds(abase + CH, CH)])
    pltpu.sync_copy(rows0.at[pl.ds(0, ROWS - 2 * CH)],
                    acc.at[pl.ds(abase + 2 * CH, ROWS - 2 * CH)])

    pltpu.sync_copy(offs_hbm, offv)
    ov = offv[pl.ds(wid, 16)]
    start = ov[0]
    end = ov[1]

    c0 = start // CH
    c1 = (end + CH - 1) // CH
    n = c1 - c0

    def fetch_idx(b, i):
        pltpu.async_copy(edata_hbm.at[c0 + i], ibs[b], isem[b])
        pltpu.async_copy(wdat_hbm.at[c0 + i], wbs[b], isem[b])

    def start_gather(b):
        pltpu.make_async_copy(edata_hbm.at[c0], ibs[b], isem[b]).wait()
        pltpu.make_async_copy(wdat_hbm.at[c0], wbs[b], isem[b]).wait()
        pltpu.async_copy(feat_hbm.at[ibs[b].at[0]], rows[b], gsem[b])

    def consume(b, i):
        pltpu.make_async_copy(feat_hbm.at[ibs[b].at[0]], rows[b],
                              gsem[b]).wait()
        bn = (b + 1) % NBUF

        @pl.when(i + 1 < n)
        def _():
            start_gather(bn)

        base = (c0 + i) * CH

        def _scale_group(g, _):
            gb = g * 16
            wvec = wbs[b][pl.ds(gb, 16)]
            lane = base + gb + lax.iota(jnp.int32, 16)
            wvec = jnp.where((lane >= start) & (lane < end), wvec, 0.0)
            for e in range(16):
                wb = wvec[jnp.broadcast_to(jnp.int32(e), (16,))]
                r = gb + e
                for j in range(HID // 16):
                    rows[b][r, pl.ds(j * 16, 16)] = (
                        rows[b][r, pl.ds(j * 16, 16)] * wb)
            return 0

        lax.fori_loop(0, CH // 16, _scale_group, 0)
        pltpu.async_copy(rows[b], acc.at[ibs[b].at[1]], ssem[b], add=True)

    def wait_scatter(b):
        pltpu.make_async_copy(rows[b], acc.at[ibs[b].at[1]], ssem[b]).wait()

    @pl.when(0 < n)
    def _():
        fetch_idx(0, 0)

        @pl.when(1 < n)
        def _():
            fetch_idx(1, 1)
        start_gather(0)

    def body(jj, _):
        i0 = jj * NBUF
        for b in range(NBUF):
            i = i0 + b
            br = (b + 2) % NBUF

            @pl.when(i < n)
            def _(b=b, i=i, br=br):
                consume(b, i)
                k = i + 2

                @pl.when(k < n)
                def _():
                    @pl.when(k >= NBUF)
                    def _():
                        wait_scatter(br)
                    fetch_idx(br, k)
        return 0

    lax.fori_loop(0, (n + NBUF - 1) // NBUF, body, 0)

    for b in range(NBUF):
        @pl.when(b < n)
        def _(b=b):
            wait_scatter(b)

    plsc.subcore_barrier()
    pltpu.sync_copy(acc.at[pl.ds(sid * ROWS, ROWS)],
                    out_hbm.at[pl.ds(wid * ROWS, ROWS)])


@jax.jit
def _diffusion_step(feat, edata, wdat, offs):
    mesh = plsc.VectorSubcoreMesh(core_axis_name="c", subcore_axis_name="s",
                                  num_cores=2, num_subcores=16)
    return pl.kernel(
        _diffusion_body,
        out_type=jax.ShapeDtypeStruct((NPAD, HID), jnp.float32),
        mesh=mesh,
        scratch_types=[
            pltpu.VMEM_SHARED((16 * ROWS, HID), jnp.float32),
            pltpu.VMEM((CH, HID), jnp.float32),
            pltpu.VMEM((CH, HID), jnp.float32),
            pltpu.VMEM((CH, HID), jnp.float32),
            pltpu.VMEM((2, CH), jnp.int32),
            pltpu.VMEM((2, CH), jnp.int32),
            pltpu.VMEM((2, CH), jnp.int32),
            pltpu.VMEM((CH,), jnp.float32),
            pltpu.VMEM((CH,), jnp.float32),
            pltpu.VMEM((CH,), jnp.float32),
            pltpu.VMEM((NOFF,), jnp.int32),
            pltpu.SemaphoreType.DMA,
            pltpu.SemaphoreType.DMA,
            pltpu.SemaphoreType.DMA,
            pltpu.SemaphoreType.DMA,
            pltpu.SemaphoreType.DMA,
            pltpu.SemaphoreType.DMA,
            pltpu.SemaphoreType.DMA,
            pltpu.SemaphoreType.DMA,
            pltpu.SemaphoreType.DMA,
        ],
    )(feat, edata, wdat, offs)


RBLK = 1280
GRID = NPAD // RBLK


def _px_body(wx_ref, b_ref, x0_ref, xch_ref, out_ref):
    acc = jnp.broadcast_to(b_ref[0, :], (RBLK, HID))
    acc = acc + jnp.dot(x0_ref[...], wx_ref[0],
                        preferred_element_type=jnp.float32)
    for k in range(K):
        acc = acc + jnp.dot(xch_ref[k], wx_ref[k + 1],
                            preferred_element_type=jnp.float32)
    out_ref[...] = acc


def _px_call(wx, b2, x0, xch):
    blk = pl.BlockSpec((RBLK, HID), lambda i: (i, 0))
    chblk = pl.BlockSpec((K, RBLK, HID), lambda i: (0, i, 0))
    return pl.pallas_call(
        _px_body,
        grid=(GRID,),
        in_specs=[pl.BlockSpec((K + 1, HID, HID), lambda i: (0, 0, 0)),
                  pl.BlockSpec((1, HID), lambda i: (0, 0)), blk, chblk],
        out_specs=blk,
        out_shape=jax.ShapeDtypeStruct((NPAD, HID), jnp.float32),
    )(wx, b2, x0, xch)


def _gate_body(wh_ref, px_ref, h_ref, hch_ref, ru_ref, rh_ref):
    acc = px_ref[...]
    acc = acc + jnp.dot(h_ref[...], wh_ref[0],
                        preferred_element_type=jnp.float32)
    for k in range(K):
        acc = acc + jnp.dot(hch_ref[k], wh_ref[k + 1],
                            preferred_element_type=jnp.float32)
    ru = jax.nn.sigmoid(acc)
    ru_ref[...] = ru
    rh_ref[...] = ru * h_ref[...]


def _gate_call(wh, px, h, hch):
    blk = pl.BlockSpec((RBLK, HID), lambda i: (i, 0))
    chblk = pl.BlockSpec((K, RBLK, HID), lambda i: (0, i, 0))
    return pl.pallas_call(
        _gate_body,
        grid=(GRID,),
        in_specs=[pl.BlockSpec((K + 1, HID, HID), lambda i: (0, 0, 0)),
                  blk, blk, chblk],
        out_specs=[blk, blk],
        out_shape=[jax.ShapeDtypeStruct((NPAD, HID), jnp.float32),
                   jax.ShapeDtypeStruct((NPAD, HID), jnp.float32)],
    )(wh, px, h, hch)


def _cand_body(wh_ref, px_ref, h_ref, ru_ref, rh_ref, rhch_ref, out_ref):
    acc = px_ref[...]
    acc = acc + jnp.dot(rh_ref[...], wh_ref[0],
                        preferred_element_type=jnp.float32)
    for k in range(K):
        acc = acc + jnp.dot(rhch_ref[k], wh_ref[k + 1],
                            preferred_element_type=jnp.float32)
    c = jax.nn.sigmoid(acc)
    ru = ru_ref[...]
    out_ref[...] = ru * h_ref[...] + (1.0 - ru) * c


def _cand_call(wh, px, h, ru, rh, rhch):
    blk = pl.BlockSpec((RBLK, HID), lambda i: (i, 0))
    chblk = pl.BlockSpec((K, RBLK, HID), lambda i: (0, i, 0))
    return pl.pallas_call(
        _cand_body,
        grid=(GRID,),
        in_specs=[pl.BlockSpec((K + 1, HID, HID), lambda i: (0, 0, 0)),
                  blk, blk, blk, blk, chblk],
        out_specs=blk,
        out_shape=jax.ShapeDtypeStruct((NPAD, HID), jnp.float32),
    )(wh, px, h, ru, rh, rhch)


def kernel(input, hidden, edge_index, edge_weight, W, b):
    src, dst = edge_index[0], edge_index[1]

    order = jnp.argsort(dst)
    dsts = dst[order]
    srcs = src[order]
    wdat = edge_weight[order].reshape(NCH, CH)
    dstl = (dsts % (16 * ROWS)).astype(jnp.int32)
    edata = jnp.stack([srcs.reshape(NCH, CH), dstl.reshape(NCH, CH)], axis=1)
    bounds = (jnp.arange(NOFF, dtype=jnp.int32) * ROWS).clip(max=NPAD)
    offs = jnp.searchsorted(dsts, bounds).astype(jnp.int32)

    w3 = W.reshape(K + 1, IN + HID, HID)
    wx = w3[:, :IN, :]
    wh = w3[:, IN:, :]
    b2 = b.reshape(1, HID)

    pad_n = ((0, NPAD - N), (0, 0))
    xs4 = jnp.pad(input, ((0, 0),) + pad_n)
    h0 = jnp.pad(hidden[0], pad_n)

    def chain(feat0):
        def body(f, _):
            fn = _diffusion_step(f, edata, wdat, offs)
            return fn, fn
        _, ys = lax.scan(body, feat0, None, length=K)
        return ys

    def px_step(_, x0):
        xch = chain(x0)
        return 0, _px_call(wx, b2, x0, xch)

    _, pxs = lax.scan(px_step, 0, xs4)

    def tstep(h, px_t):
        hch = chain(h)
        ru, rh = _gate_call(wh, px_t, h, hch)
        rhch = chain(rh)
        hn = _cand_call(wh, px_t, h, ru, rh, rhch)
        return hn, hn

    h_fin, outs = lax.scan(tstep, h0, pxs)

    output = outs[:, :N, :]
    hidden_out = h_fin[:N][None, :, :]
    return (output, hidden_out)
